# baseline (speedup 1.0000x reference)
import functools

import jax
import jax.numpy as jnp
from jax import lax
from jax.experimental import pallas as pl
from jax.experimental.pallas import tpu as pltpu
from jax.experimental.pallas import tpu_sc as plsc

_NC = 2
_NS = 16
_NW = _NC * _NS
_C = 64


@functools.cache
def _make_gather(B, V, D):
    BPW = B // _NW
    NCHUNK = BPW // _C
    mesh = plsc.VectorSubcoreMesh(core_axis_name="c", subcore_axis_name="s")

    NBUF = 4
    LEAD = 2

    @functools.partial(
        pl.kernel,
        out_type=jax.ShapeDtypeStruct((B, D), jnp.float32),
        mesh=mesh,
        scratch_types=[
            pltpu.VMEM((NCHUNK, _C), jnp.int32),
            pltpu.VMEM((NBUF, _C, D), jnp.float32),
            [pltpu.SemaphoreType.DMA] * NBUF,
            [pltpu.SemaphoreType.DMA] * NBUF,
        ],
    )
    def gather_kernel(idx_hbm, table_hbm, out_hbm, idx_v, rows_v, gsems, osems):
        wid = lax.axis_index("s") * _NC + lax.axis_index("c")
        base = wid * BPW
        pltpu.sync_copy(idx_hbm.at[wid], idx_v)

        def gather(g, b):
            pltpu.async_copy(table_hbm.at[idx_v.at[g]], rows_v.at[b], gsems[b])

        def wait_gather(g, b):
            pltpu.make_async_copy(
                table_hbm.at[idx_v.at[g]], rows_v.at[b], gsems[b]
            ).wait()

        def writeback(g, b):
            pltpu.async_copy(
                rows_v.at[b], out_hbm.at[pl.ds(base + g * _C, _C)], osems[b]
            )

        def wait_writeback(g, b):
            pltpu.make_async_copy(
                rows_v.at[b], out_hbm.at[pl.ds(base + g * _C, _C)], osems[b]
            ).wait()

        for b in range(LEAD):
            gather(b, b)

        def outer(i, carry):
            g0 = i * NBUF
            for b in range(NBUF):
                g = g0 + b
                wait_gather(g, b)
                writeback(g, b)
                nb = (b + LEAD) % NBUF

                @pl.when(g + LEAD - NBUF >= 0)
                def _():
                    wait_writeback(g + LEAD - NBUF, nb)

                @pl.when(g + LEAD < NCHUNK)
                def _():
                    gather(g + LEAD, nb)

            return carry

        lax.fori_loop(0, NCHUNK // NBUF, outer, 0)

        for k in range(NBUF - LEAD):
            g = NCHUNK - NBUF + LEAD + k
            wait_writeback(g, (LEAD + k) % NBUF)

    return gather_kernel


def kernel(x, table):
    B0, B1 = x.shape
    V, D = table.shape
    B = B0 * B1
    idx = x.reshape(_NW, (B // _NW) // _C, _C).astype(jnp.int32)
    out = _make_gather(B, V, D)(idx, table)
    return out.reshape(B0, B1, D)


# speedup vs baseline: 1.0051x; 1.0051x over previous
import functools

import jax
import jax.numpy as jnp
from jax import lax
from jax.experimental import pallas as pl
from jax.experimental.pallas import tpu as pltpu
from jax.experimental.pallas import tpu_sc as plsc

_NC = 2
_NS = 16
_NW = _NC * _NS
_C = 128
_NBUF = 2


@functools.cache
def _make_gather(B, V, D):
    BPW = B // _NW
    NCHUNK = BPW // _C
    mesh = plsc.VectorSubcoreMesh(core_axis_name="c", subcore_axis_name="s")

    @functools.partial(
        pl.kernel,
        out_type=jax.ShapeDtypeStruct((B, D), jnp.float32),
        mesh=mesh,
        scratch_types=[
            pltpu.VMEM((NCHUNK, _C), jnp.int32),
            pltpu.VMEM((_NBUF, _C, D), jnp.float32),
            [pltpu.SemaphoreType.DMA] * _NBUF,
        ],
    )
    def gather_kernel(idx_hbm, table_hbm, out_hbm, idx_v, rows_v, gsems):
        wid = lax.axis_index("s") * _NC + lax.axis_index("c")
        base = wid * BPW
        pltpu.sync_copy(idx_hbm.at[wid], idx_v)

        for b in range(_NBUF):
            pltpu.async_copy(table_hbm.at[idx_v.at[b]], rows_v.at[b], gsems[b])

        def outer(i, carry):
            g0 = i * _NBUF
            for b in range(_NBUF):
                g = g0 + b
                pltpu.make_async_copy(
                    table_hbm.at[idx_v.at[g]], rows_v.at[b], gsems[b]
                ).wait()
                pltpu.sync_copy(rows_v.at[b], out_hbm.at[pl.ds(base + g * _C, _C)])
                ng = g + _NBUF

                @pl.when(ng < NCHUNK)
                def _():
                    pltpu.async_copy(
                        table_hbm.at[idx_v.at[ng]], rows_v.at[b], gsems[b]
                    )

            return carry

        lax.fori_loop(0, NCHUNK // _NBUF, outer, 0)

    return gather_kernel


def kernel(x, table):
    B0, B1 = x.shape
    V, D = table.shape
    B = B0 * B1
    idx = x.reshape(_NW, (B // _NW) // _C, _C).astype(jnp.int32)
    out = _make_gather(B, V, D)(idx, table)
    return out.reshape(B0, B1, D)
